# baseline (speedup 1.0000x reference)
import functools

import jax
import jax.numpy as jnp
from jax import lax
from jax.experimental import pallas as pl
from jax.experimental.pallas import tpu as pltpu
from jax.experimental.pallas import tpu_sc as plsc

_N = 10000
_E = 320000
_D = 128
_G = 64
_C = 10

_NC = 2
_NS = 16
_NW = _NC * _NS
_WIN = 128
_NWIN = 80
_EPW = _WIN * _NWIN
_EP = _NW * _EPW
_GW = 16
_NG = _NWIN // _GW
_NP = 10240
_RPS = _NP // _NS


def _seg_sum_sc(x, src3, dst3, zeros_blk):
    mesh = plsc.VectorSubcoreMesh(core_axis_name="c", subcore_axis_name="s")

    @functools.partial(
        pl.kernel,
        mesh=mesh,
        out_type=jax.ShapeDtypeStruct((_NC, _NP, _D), jnp.float32),
        scratch_types=[
            pltpu.VMEM((_GW, _WIN), jnp.int32),
            pltpu.VMEM((_GW, _WIN), jnp.int32),
            pltpu.VMEM((_GW, _WIN), jnp.int32),
            pltpu.VMEM((_GW, _WIN), jnp.int32),
            pltpu.VMEM((_WIN, _D), jnp.float32),
            pltpu.VMEM((_WIN, _D), jnp.float32),
            pltpu.VMEM_SHARED((_NP, _D), jnp.float32),
            pltpu.SemaphoreType.DMA,
            pltpu.SemaphoreType.DMA,
            pltpu.SemaphoreType.DMA,
            pltpu.SemaphoreType.DMA,
        ],
    )
    def seg_kernel(x_hbm, src_hbm, dst_hbm, zero_hbm, out_hbm,
                   src_va, dst_va, src_vb, dst_vb, rows_a, rows_b, acc,
                   sem_a, sem_b, sem_i, sem_z):
        c = lax.axis_index("c")
        s = lax.axis_index("s")
        wid = s * _NC + c

        def _gather(idx_ref, j, buf, sem):
            pltpu.async_copy(x_hbm.at[idx_ref.at[j]], buf, sem)

        def _gwait(buf, sem):
            pltpu.make_async_copy(x_hbm.at[pl.ds(0, _WIN)], buf, sem).wait()

        zc = pltpu.async_copy(zero_hbm, acc.at[pl.ds(s * _RPS, _RPS)],
                              sem_z)
        pltpu.sync_copy(src_hbm.at[wid, pl.ds(0, _GW)], src_va)
        pltpu.sync_copy(dst_hbm.at[wid, pl.ds(0, _GW)], dst_va)
        _gather(src_va, 0, rows_a, sem_a)
        _gather(src_va, 1, rows_b, sem_b)
        zc.wait()
        plsc.subcore_barrier()

        for g in range(_NG):
            src_cur, dst_cur = (src_va, dst_va) if g % 2 == 0 else (src_vb,
                                                                    dst_vb)
            src_nxt, dst_nxt = (src_vb, dst_vb) if g % 2 == 0 else (src_va,
                                                                    dst_va)
            last = g + 1 == _NG
            if not last:
                ci = pltpu.async_copy(
                    src_hbm.at[wid, pl.ds((g + 1) * _GW, _GW)], src_nxt,
                    sem_i)
                cj = pltpu.async_copy(
                    dst_hbm.at[wid, pl.ds((g + 1) * _GW, _GW)], dst_nxt,
                    sem_i)

            _gwait(rows_a, sem_a)
            pltpu.sync_copy(rows_a, acc.at[dst_cur.at[0]], add=True)
            _gather(src_cur, 2, rows_a, sem_a)
            _gwait(rows_b, sem_b)
            pltpu.sync_copy(rows_b, acc.at[dst_cur.at[1]], add=True)
            _gather(src_cur, 3, rows_b, sem_b)
            if not last:
                ci.wait()
                cj.wait()

            @pl.loop(2, _GW, step=2)
            def _main(j, src_cur=src_cur, dst_cur=dst_cur,
                      src_nxt=src_nxt, last=last):
                _gwait(rows_a, sem_a)
                pltpu.sync_copy(rows_a, acc.at[dst_cur.at[j]], add=True)

                @pl.when(j + 2 < _GW)
                def _():
                    _gather(src_cur, j + 2, rows_a, sem_a)

                if not last:
                    @pl.when(j + 2 >= _GW)
                    def _():
                        _gather(src_nxt, j + 2 - _GW, rows_a, sem_a)

                _gwait(rows_b, sem_b)
                pltpu.sync_copy(rows_b, acc.at[dst_cur.at[j + 1]], add=True)

                @pl.when(j + 3 < _GW)
                def _():
                    _gather(src_cur, j + 3, rows_b, sem_b)

                if not last:
                    @pl.when(j + 3 >= _GW)
                    def _():
                        _gather(src_nxt, j + 3 - _GW, rows_b, sem_b)

        plsc.subcore_barrier()

        pltpu.sync_copy(acc.at[pl.ds(s * _RPS, _RPS)],
                        out_hbm.at[c, pl.ds(s * _RPS, _RPS)])

    return seg_kernel(x, src3, dst3, zeros_blk)


def _dot(a, b):
    return jnp.dot(a, b, preferred_element_type=jnp.float32)


def _bn_relu(h, g, be):
    m = jnp.mean(h, axis=0, keepdims=True)
    v = jnp.mean((h - m) ** 2, axis=0, keepdims=True)
    return jnp.maximum((h - m) * lax.rsqrt(v + 1e-5) * g + be, 0.0)


_B = 5000
_NB = _N // _B


def _row_spec(ndim=2, pin=False):
    if pin:
        def idx(ph, j):
            return j + ph * (_NB - 1 - j)
    else:
        def idx(ph, j):
            return j
    if ndim == 2:
        return pl.BlockSpec((_B, _D), lambda ph, j: (idx(ph, j), 0))
    return pl.BlockSpec((_NC, _B, _D), lambda ph, j: (0, idx(ph, j), 0))


def _full_spec(shape):
    return pl.BlockSpec(shape, lambda ph, j: tuple(0 for _ in shape))


def _gin_block_tc(xp, parts, W, b, eps, g, be, res):
    def body(x_ref, p_ref, w_ref, b_ref, eps_ref, g_ref, be_ref, o_ref,
             h_scr, stat_scr):
        ph = pl.program_id(0)
        j = pl.program_id(1)

        @pl.when(ph == 0)
        def _phase0():
            agg = p_ref[0] + p_ref[1]
            z = x_ref[...] * (1.0 + eps_ref[...]) + agg
            h = _dot(z, w_ref[...]) + b_ref[...]
            h_scr[pl.ds(j * _B, _B), :] = h
            s0 = jnp.sum(h, axis=0, keepdims=True)
            s1 = jnp.sum(h * h, axis=0, keepdims=True)

            @pl.when(j == 0)
            def _():
                stat_scr[0:1, :] = s0
                stat_scr[1:2, :] = s1

            @pl.when(j > 0)
            def _():
                stat_scr[0:1, :] += s0
                stat_scr[1:2, :] += s1

        @pl.when(ph == 1)
        def _phase1():
            m = stat_scr[0:1, :] * (1.0 / _N)
            v = stat_scr[1:2, :] * (1.0 / _N) - m * m
            h = h_scr[pl.ds(j * _B, _B), :]
            hn = (h - m) * lax.rsqrt(v + 1e-5) * g_ref[...] + be_ref[...]
            hn = jnp.maximum(hn, 0.0)
            if res:
                hn = hn + x_ref[...]
            o_ref[...] = hn

    args = [xp, parts, W, b.reshape(1, _D), eps.reshape(1, 1),
            g.reshape(1, _D), be.reshape(1, _D)]
    in_specs = [_row_spec(pin=not res), _row_spec(3, pin=True),
                _full_spec((_D, _D)), _full_spec((1, _D)),
                _full_spec((1, 1)), _full_spec((1, _D)),
                _full_spec((1, _D))]
    return pl.pallas_call(
        body,
        grid=(2, _NB),
        in_specs=in_specs,
        out_specs=_row_spec(),
        out_shape=jax.ShapeDtypeStruct((_N, _D), jnp.float32),
        scratch_shapes=[pltpu.VMEM((_N, _D), jnp.float32),
                        pltpu.VMEM((8, _D), jnp.float32)],
    )(*args)


def _t_term_tc(xp, parts, eps4):
    def body(x_ref, p_ref, eps4_ref, t_ref):
        t_ref[...] = (x_ref[...] * (1.0 + eps4_ref[...])
                      + p_ref[0] + p_ref[1])

    return pl.pallas_call(
        body,
        grid=(_NB,),
        in_specs=[pl.BlockSpec((_B, _D), lambda j: (j, 0)),
                  pl.BlockSpec((_NC, _B, _D), lambda j: (0, j, 0)),
                  pl.BlockSpec((1, 1), lambda j: (0, 0))],
        out_specs=pl.BlockSpec((_B, _D), lambda j: (j, 0)),
        out_shape=jax.ShapeDtypeStruct((_N, _D), jnp.float32),
    )(xp, parts, eps4.reshape(1, 1))


def _block4_pool_tc(t1, t2, x3, p3, W4, b4, eps4, g4, be4, batch3d, Wh, bh):
    def body(t1_ref, t2_ref, x3_ref, p_ref, w_ref, b_ref, eps_ref, g_ref,
             be_ref, batch_ref, wh_ref, bh_ref, o_ref, h_scr, stat_scr,
             pool_scr):
        ph = pl.program_id(0)
        j = pl.program_id(1)

        @pl.when(ph == 0)
        def _phase0():
            z3 = x3_ref[...] * (1.0 + eps_ref[...]) + p_ref[0] + p_ref[1]
            w = w_ref[...]
            h = (_dot(t1_ref[...], w[0:_D])
                 + _dot(t2_ref[...], w[_D:2 * _D])
                 + _dot(z3, w[2 * _D:3 * _D]) + b_ref[...])
            h_scr[pl.ds(j * _B, _B), :] = h
            s0 = jnp.sum(h, axis=0, keepdims=True)
            s1 = jnp.sum(h * h, axis=0, keepdims=True)

            @pl.when(j == 0)
            def _():
                stat_scr[0:1, :] = s0
                stat_scr[1:2, :] = s1

            @pl.when(j > 0)
            def _():
                stat_scr[0:1, :] += s0
                stat_scr[1:2, :] += s1

        @pl.when(ph == 1)
        def _phase1():
            m = stat_scr[0:1, :] * (1.0 / _N)
            v = stat_scr[1:2, :] * (1.0 / _N) - m * m
            h = h_scr[pl.ds(j * _B, _B), :]
            hn = (h - m) * lax.rsqrt(v + 1e-5) * g_ref[...] + be_ref[...]
            x4 = jnp.maximum(hn, 0.0)
            gids = lax.broadcasted_iota(jnp.int32, (_G, _B), 0)
            mask = (gids == batch_ref[0]).astype(jnp.float32)
            pooled = _dot(mask, x4)

            @pl.when(j == 0)
            def _():
                pool_scr[...] = pooled

            @pl.when(j > 0)
            def _():
                pool_scr[...] += pooled

            @pl.when(j == _NB - 1)
            def _():
                logits = _dot(pool_scr[...], wh_ref[...]) + bh_ref[...]
                mx = jnp.max(logits, axis=-1, keepdims=True)
                lse = jnp.log(jnp.sum(jnp.exp(logits - mx), axis=-1,
                                      keepdims=True)) + mx
                o_ref[...] = logits - lse

    return pl.pallas_call(
        body,
        grid=(2, _NB),
        in_specs=[_row_spec(pin=True), _row_spec(pin=True),
                  _row_spec(pin=True), _row_spec(3, pin=True),
                  _full_spec((3 * _D, _D)), _full_spec((1, _D)),
                  _full_spec((1, 1)), _full_spec((1, _D)),
                  _full_spec((1, _D)),
                  pl.BlockSpec((1, 1, _B), lambda ph, j: (j, 0, 0)),
                  _full_spec((_D, _C)), _full_spec((1, _C))],
        out_specs=pl.BlockSpec((_G, _C), lambda ph, j: (0, 0)),
        out_shape=jax.ShapeDtypeStruct((_G, _C), jnp.float32),
        scratch_shapes=[pltpu.VMEM((_N, _D), jnp.float32),
                        pltpu.VMEM((8, _D), jnp.float32),
                        pltpu.VMEM((_G, _D), jnp.float32)],
    )(t1, t2, x3, p3, W4, b4.reshape(1, _D), eps4.reshape(1, 1),
      g4.reshape(1, _D), be4.reshape(1, _D), batch3d, Wh,
      bh.reshape(1, _C))


def kernel(x, edge_index, batch,
           W1, b1, eps1, g1, be1,
           W2, b2, eps2, g2, be2,
           W3, b3, eps3, g3, be3,
           W4, b4, eps4, g4, be4,
           Wh, bh):
    padi = jnp.arange(_EP - _E, dtype=jnp.int32)
    src3 = jnp.concatenate([edge_index[0], padi % _N]).reshape(
        _NW, _NWIN, _WIN)
    dst3 = jnp.concatenate([edge_index[1], _N + padi % (_NP - _N)]).reshape(
        _NW, _NWIN, _WIN)
    zeros_blk = jnp.zeros((_RPS, _D), jnp.float32)

    p0 = _seg_sum_sc(x, src3, dst3, zeros_blk)
    x1 = _gin_block_tc(x, p0, W1, b1, eps1, g1, be1, res=False)
    p1 = _seg_sum_sc(x1, src3, dst3, zeros_blk)
    x2 = _gin_block_tc(x1, p1, W2, b2, eps2, g2, be2, res=True)
    t1 = _t_term_tc(x1, p1, eps4)
    p2 = _seg_sum_sc(x2, src3, dst3, zeros_blk)
    x3 = _gin_block_tc(x2, p2, W3, b3, eps3, g3, be3, res=True)
    t2 = _t_term_tc(x2, p2, eps4)
    p3 = _seg_sum_sc(x3, src3, dst3, zeros_blk)
    return _block4_pool_tc(t1, t2, x3, p3, W4, b4, eps4, g4, be4,
                           batch.reshape(_NB, 1, _B), Wh, bh)


# speedup vs baseline: 1.0086x; 1.0086x over previous
import functools

import jax
import jax.numpy as jnp
from jax import lax
from jax.experimental import pallas as pl
from jax.experimental.pallas import tpu as pltpu
from jax.experimental.pallas import tpu_sc as plsc

_N = 10000
_E = 320000
_D = 128
_G = 64
_C = 10

_NC = 2
_NS = 16
_NW = _NC * _NS
_WIN = 128
_NWIN = 80
_EPW = _WIN * _NWIN
_EP = _NW * _EPW
_GW = 16
_NG = _NWIN // _GW
_NP = 10240
_RPS = _NP // _NS


def _seg_sum_sc(x, src3, dst3, zeros_blk):
    mesh = plsc.VectorSubcoreMesh(core_axis_name="c", subcore_axis_name="s")

    @functools.partial(
        pl.kernel,
        mesh=mesh,
        out_type=jax.ShapeDtypeStruct((_NC, _NP, _D), jnp.float32),
        scratch_types=[
            pltpu.VMEM((_GW, _WIN), jnp.int32),
            pltpu.VMEM((_GW, _WIN), jnp.int32),
            pltpu.VMEM((_GW, _WIN), jnp.int32),
            pltpu.VMEM((_GW, _WIN), jnp.int32),
            pltpu.VMEM((_WIN, _D), jnp.float32),
            pltpu.VMEM((_WIN, _D), jnp.float32),
            pltpu.VMEM_SHARED((_NP, _D), jnp.float32),
            pltpu.SemaphoreType.DMA,
            pltpu.SemaphoreType.DMA,
            pltpu.SemaphoreType.DMA,
            pltpu.SemaphoreType.DMA,
        ],
    )
    def seg_kernel(x_hbm, src_hbm, dst_hbm, zero_hbm, out_hbm,
                   src_va, dst_va, src_vb, dst_vb, rows_a, rows_b, acc,
                   sem_a, sem_b, sem_i, sem_z):
        c = lax.axis_index("c")
        s = lax.axis_index("s")
        wid = s * _NC + c

        def _gather(idx_ref, j, buf, sem):
            pltpu.async_copy(x_hbm.at[idx_ref.at[j]], buf, sem)

        def _gwait(buf, sem):
            pltpu.make_async_copy(x_hbm.at[pl.ds(0, _WIN)], buf, sem).wait()

        zc = pltpu.async_copy(zero_hbm, acc.at[pl.ds(s * _RPS, _RPS)],
                              sem_z)
        pltpu.sync_copy(src_hbm.at[wid, pl.ds(0, _GW)], src_va)
        pltpu.sync_copy(dst_hbm.at[wid, pl.ds(0, _GW)], dst_va)
        _gather(src_va, 0, rows_a, sem_a)
        _gather(src_va, 1, rows_b, sem_b)
        zc.wait()
        plsc.subcore_barrier()

        for g in range(_NG):
            src_cur, dst_cur = (src_va, dst_va) if g % 2 == 0 else (src_vb,
                                                                    dst_vb)
            src_nxt, dst_nxt = (src_vb, dst_vb) if g % 2 == 0 else (src_va,
                                                                    dst_va)
            last = g + 1 == _NG
            if not last:
                ci = pltpu.async_copy(
                    src_hbm.at[wid, pl.ds((g + 1) * _GW, _GW)], src_nxt,
                    sem_i)
                cj = pltpu.async_copy(
                    dst_hbm.at[wid, pl.ds((g + 1) * _GW, _GW)], dst_nxt,
                    sem_i)

            _gwait(rows_a, sem_a)
            pltpu.sync_copy(rows_a, acc.at[dst_cur.at[0]], add=True)
            _gather(src_cur, 2, rows_a, sem_a)
            _gwait(rows_b, sem_b)
            pltpu.sync_copy(rows_b, acc.at[dst_cur.at[1]], add=True)
            _gather(src_cur, 3, rows_b, sem_b)
            if not last:
                ci.wait()
                cj.wait()

            @pl.loop(2, _GW, step=2)
            def _main(j, src_cur=src_cur, dst_cur=dst_cur,
                      src_nxt=src_nxt, last=last):
                _gwait(rows_a, sem_a)
                pltpu.sync_copy(rows_a, acc.at[dst_cur.at[j]], add=True)

                @pl.when(j + 2 < _GW)
                def _():
                    _gather(src_cur, j + 2, rows_a, sem_a)

                if not last:
                    @pl.when(j + 2 >= _GW)
                    def _():
                        _gather(src_nxt, j + 2 - _GW, rows_a, sem_a)

                _gwait(rows_b, sem_b)
                pltpu.sync_copy(rows_b, acc.at[dst_cur.at[j + 1]], add=True)

                @pl.when(j + 3 < _GW)
                def _():
                    _gather(src_cur, j + 3, rows_b, sem_b)

                if not last:
                    @pl.when(j + 3 >= _GW)
                    def _():
                        _gather(src_nxt, j + 3 - _GW, rows_b, sem_b)

        plsc.subcore_barrier()

        pltpu.sync_copy(acc.at[pl.ds(s * _RPS, _RPS)],
                        out_hbm.at[c, pl.ds(s * _RPS, _RPS)])

    return seg_kernel(x, src3, dst3, zeros_blk)


def _dot(a, b):
    return jnp.dot(a, b, preferred_element_type=jnp.float32)


def _bn_relu(h, g, be):
    m = jnp.mean(h, axis=0, keepdims=True)
    v = jnp.mean((h - m) ** 2, axis=0, keepdims=True)
    return jnp.maximum((h - m) * lax.rsqrt(v + 1e-5) * g + be, 0.0)


_B = 5000
_NB = _N // _B


def _row_spec(ndim=2, pin=False):
    if pin:
        def idx(ph, j):
            return j + ph * (_NB - 1 - j)
    else:
        def idx(ph, j):
            return j
    if ndim == 2:
        return pl.BlockSpec((_B, _D), lambda ph, j: (idx(ph, j), 0))
    return pl.BlockSpec((_NC, _B, _D), lambda ph, j: (0, idx(ph, j), 0))


def _full_spec(shape):
    return pl.BlockSpec(shape, lambda ph, j: tuple(0 for _ in shape))


def _gin_block_tc(xp, parts, W, b, eps, g, be, res, eps4=None):
    emit_t = eps4 is not None
    outs = [jax.ShapeDtypeStruct((_N, _D), jnp.float32)]
    if emit_t:
        outs.append(jax.ShapeDtypeStruct((_N, _D), jnp.float32))

    def body(x_ref, p_ref, w_ref, b_ref, eps_ref, g_ref, be_ref, *rest):
        if emit_t:
            eps4_ref = rest[0]
            o_ref, t_ref, h_scr, agg_scr, stat_scr = rest[1:]
        else:
            o_ref, h_scr, agg_scr, stat_scr = rest
        ph = pl.program_id(0)
        j = pl.program_id(1)

        @pl.when(ph == 0)
        def _phase0():
            agg = p_ref[0] + p_ref[1]
            if emit_t:
                agg_scr[pl.ds(j * _B, _B), :] = agg
            z = x_ref[...] * (1.0 + eps_ref[...]) + agg
            h = _dot(z, w_ref[...]) + b_ref[...]
            h_scr[pl.ds(j * _B, _B), :] = h
            s0 = jnp.sum(h, axis=0, keepdims=True)
            s1 = jnp.sum(h * h, axis=0, keepdims=True)

            @pl.when(j == 0)
            def _():
                stat_scr[0:1, :] = s0
                stat_scr[1:2, :] = s1

            @pl.when(j > 0)
            def _():
                stat_scr[0:1, :] += s0
                stat_scr[1:2, :] += s1

        @pl.when(ph == 1)
        def _phase1():
            m = stat_scr[0:1, :] * (1.0 / _N)
            v = stat_scr[1:2, :] * (1.0 / _N) - m * m
            h = h_scr[pl.ds(j * _B, _B), :]
            hn = (h - m) * lax.rsqrt(v + 1e-5) * g_ref[...] + be_ref[...]
            hn = jnp.maximum(hn, 0.0)
            if res:
                hn = hn + x_ref[...]
            o_ref[...] = hn
            if emit_t:
                t_ref[...] = (x_ref[...] * (1.0 + eps4_ref[...])
                              + agg_scr[pl.ds(j * _B, _B), :])

    args = [xp, parts, W, b.reshape(1, _D), eps.reshape(1, 1),
            g.reshape(1, _D), be.reshape(1, _D)]
    in_specs = [_row_spec(pin=not (res or emit_t)), _row_spec(3, pin=True),
                _full_spec((_D, _D)), _full_spec((1, _D)),
                _full_spec((1, 1)), _full_spec((1, _D)),
                _full_spec((1, _D))]
    if emit_t:
        args.append(eps4.reshape(1, 1))
        in_specs.append(_full_spec((1, 1)))
    out_specs = [_row_spec()] * (2 if emit_t else 1)
    agg_rows = _N if emit_t else 8
    return pl.pallas_call(
        body,
        grid=(2, _NB),
        in_specs=in_specs,
        out_specs=out_specs if emit_t else out_specs[0],
        out_shape=outs if emit_t else outs[0],
        scratch_shapes=[pltpu.VMEM((_N, _D), jnp.float32),
                        pltpu.VMEM((agg_rows, _D), jnp.float32),
                        pltpu.VMEM((8, _D), jnp.float32)],
    )(*args)


def _block4_pool_tc(t1, t2, x3, p3, W4, b4, eps4, g4, be4, batch3d, Wh, bh):
    def body(t1_ref, t2_ref, x3_ref, p_ref, w_ref, b_ref, eps_ref, g_ref,
             be_ref, batch_ref, wh_ref, bh_ref, o_ref, h_scr, stat_scr,
             pool_scr):
        ph = pl.program_id(0)
        j = pl.program_id(1)

        @pl.when(ph == 0)
        def _phase0():
            z3 = x3_ref[...] * (1.0 + eps_ref[...]) + p_ref[0] + p_ref[1]
            w = w_ref[...]
            h = (_dot(t1_ref[...], w[0:_D])
                 + _dot(t2_ref[...], w[_D:2 * _D])
                 + _dot(z3, w[2 * _D:3 * _D]) + b_ref[...])
            h_scr[pl.ds(j * _B, _B), :] = h
            s0 = jnp.sum(h, axis=0, keepdims=True)
            s1 = jnp.sum(h * h, axis=0, keepdims=True)

            @pl.when(j == 0)
            def _():
                stat_scr[0:1, :] = s0
                stat_scr[1:2, :] = s1

            @pl.when(j > 0)
            def _():
                stat_scr[0:1, :] += s0
                stat_scr[1:2, :] += s1

        @pl.when(ph == 1)
        def _phase1():
            m = stat_scr[0:1, :] * (1.0 / _N)
            v = stat_scr[1:2, :] * (1.0 / _N) - m * m
            h = h_scr[pl.ds(j * _B, _B), :]
            hn = (h - m) * lax.rsqrt(v + 1e-5) * g_ref[...] + be_ref[...]
            x4 = jnp.maximum(hn, 0.0)
            gids = lax.broadcasted_iota(jnp.int32, (_G, _B), 0)
            mask = (gids == batch_ref[0]).astype(jnp.float32)
            pooled = _dot(mask, x4)

            @pl.when(j == 0)
            def _():
                pool_scr[...] = pooled

            @pl.when(j > 0)
            def _():
                pool_scr[...] += pooled

            @pl.when(j == _NB - 1)
            def _():
                logits = _dot(pool_scr[...], wh_ref[...]) + bh_ref[...]
                mx = jnp.max(logits, axis=-1, keepdims=True)
                lse = jnp.log(jnp.sum(jnp.exp(logits - mx), axis=-1,
                                      keepdims=True)) + mx
                o_ref[...] = logits - lse

    return pl.pallas_call(
        body,
        grid=(2, _NB),
        in_specs=[_row_spec(pin=True), _row_spec(pin=True),
                  _row_spec(pin=True), _row_spec(3, pin=True),
                  _full_spec((3 * _D, _D)), _full_spec((1, _D)),
                  _full_spec((1, 1)), _full_spec((1, _D)),
                  _full_spec((1, _D)),
                  pl.BlockSpec((1, 1, _B), lambda ph, j: (j, 0, 0)),
                  _full_spec((_D, _C)), _full_spec((1, _C))],
        out_specs=pl.BlockSpec((_G, _C), lambda ph, j: (0, 0)),
        out_shape=jax.ShapeDtypeStruct((_G, _C), jnp.float32),
        scratch_shapes=[pltpu.VMEM((_N, _D), jnp.float32),
                        pltpu.VMEM((8, _D), jnp.float32),
                        pltpu.VMEM((_G, _D), jnp.float32)],
    )(t1, t2, x3, p3, W4, b4.reshape(1, _D), eps4.reshape(1, 1),
      g4.reshape(1, _D), be4.reshape(1, _D), batch3d, Wh,
      bh.reshape(1, _C))


def kernel(x, edge_index, batch,
           W1, b1, eps1, g1, be1,
           W2, b2, eps2, g2, be2,
           W3, b3, eps3, g3, be3,
           W4, b4, eps4, g4, be4,
           Wh, bh):
    padi = jnp.arange(_EP - _E, dtype=jnp.int32)
    src3 = jnp.concatenate([edge_index[0], padi % _N]).reshape(
        _NW, _NWIN, _WIN)
    dst3 = jnp.concatenate([edge_index[1], _N + padi % (_NP - _N)]).reshape(
        _NW, _NWIN, _WIN)
    zeros_blk = jnp.zeros((_RPS, _D), jnp.float32)

    p0 = _seg_sum_sc(x, src3, dst3, zeros_blk)
    x1 = _gin_block_tc(x, p0, W1, b1, eps1, g1, be1, res=False)
    p1 = _seg_sum_sc(x1, src3, dst3, zeros_blk)
    x2, t1 = _gin_block_tc(x1, p1, W2, b2, eps2, g2, be2, res=True,
                           eps4=eps4)
    p2 = _seg_sum_sc(x2, src3, dst3, zeros_blk)
    x3, t2 = _gin_block_tc(x2, p2, W3, b3, eps3, g3, be3, res=True,
                           eps4=eps4)
    p3 = _seg_sum_sc(x3, src3, dst3, zeros_blk)
    return _block4_pool_tc(t1, t2, x3, p3, W4, b4, eps4, g4, be4,
                           batch.reshape(_NB, 1, _B), Wh, bh)
